# baseline (speedup 1.0000x reference)
import jax
import jax.numpy as jnp
from jax.experimental import pallas as pl
from jax.experimental.pallas import tpu as pltpu

_G = 64


def _mm_body(a_ref, w_ref, o_ref):
    o_ref[...] = jnp.dot(a_ref[...], w_ref[...],
                         preferred_element_type=jnp.float32)


def _matmul(a, w, tm=256, tn=512):
    m, k = a.shape
    _, n = w.shape
    return pl.pallas_call(
        _mm_body,
        grid=(m // tm, n // tn),
        in_specs=[
            pl.BlockSpec((tm, k), lambda i, j: (i, 0)),
            pl.BlockSpec((k, tn), lambda i, j: (0, j)),
        ],
        out_specs=pl.BlockSpec((tm, tn), lambda i, j: (i, j)),
        out_shape=jax.ShapeDtypeStruct((m, n), jnp.float32),
    )(a, w)


def _edge_body(src_ref, dst_ref, xl_ref, xr_ref, att_ref, b_ref, o_ref,
               acc_ref, m_ref, s_ref):
    e = pl.program_id(0)
    ne = pl.num_programs(0)
    d = dst_ref[e]
    d_prev = dst_ref[jnp.maximum(e - 1, 0)]
    d_next = dst_ref[jnp.minimum(e + 1, ne - 1)]
    first = jnp.logical_or(e == 0, d != d_prev)
    last = jnp.logical_or(e == ne - 1, d != d_next)

    heads = att_ref.shape[0]
    xj = xl_ref[0]
    xi = xr_ref[0]
    att = att_ref[...]
    z = xj + xi
    za = jnp.where(z >= 0, z, 0.2 * z)
    alpha = jnp.sum(za * att, axis=(1, 2), keepdims=True)
    alpha = jnp.broadcast_to(alpha, (heads, 1, 128))

    neg = jnp.full((heads, 1, 128), -1e30, jnp.float32)
    m_old = jnp.where(first, neg, m_ref[...])
    s_old = jnp.where(first, jnp.zeros_like(neg), s_ref[...])
    acc_old = jnp.where(first, jnp.zeros_like(xj), acc_ref[...])

    m_new = jnp.maximum(m_old, alpha)
    corr = jnp.exp(m_old - m_new)
    w_e = jnp.exp(alpha - m_new)
    s_new = s_old * corr + w_e
    acc_new = acc_old * corr + xj * w_e
    m_ref[...] = m_new
    s_ref[...] = s_new
    acc_ref[...] = acc_new

    @pl.when(last)
    def _():
        out = jnp.mean(acc_new / s_new, axis=0)
        o_ref[0] = jnp.maximum(out + b_ref[0], 0.0)


def _gat_layer(h, src_s, dst_s, wl, wr, att, bias):
    np_, _ = h.shape
    heads, ch = att.shape
    hh = ch // 128
    lr = _matmul(h, jnp.concatenate([wl, wr], axis=1))
    xl = lr[:, :heads * ch].reshape(np_, heads, hh, 128)
    xr = lr[:, heads * ch:].reshape(np_, heads, hh, 128)
    ne = src_s.shape[0]
    out = pl.pallas_call(
        _edge_body,
        grid_spec=pltpu.PrefetchScalarGridSpec(
            num_scalar_prefetch=2,
            grid=(ne,),
            in_specs=[
                pl.BlockSpec((1, heads, hh, 128),
                             lambda e, s, d: (s[e], 0, 0, 0)),
                pl.BlockSpec((1, heads, hh, 128),
                             lambda e, s, d: (d[e], 0, 0, 0)),
                pl.BlockSpec((heads, hh, 128), lambda e, s, d: (0, 0, 0)),
                pl.BlockSpec((1, hh, 128), lambda e, s, d: (0, 0, 0)),
            ],
            out_specs=pl.BlockSpec((1, hh, 128), lambda e, s, d: (d[e], 0, 0)),
            scratch_shapes=[
                pltpu.VMEM((heads, hh, 128), jnp.float32),
                pltpu.VMEM((heads, 1, 128), jnp.float32),
                pltpu.VMEM((heads, 1, 128), jnp.float32),
            ],
        ),
        out_shape=jax.ShapeDtypeStruct((np_, hh, 128), jnp.float32),
    )(src_s, dst_s, xl, xr, att.reshape(heads, hh, 128),
      bias.reshape(1, hh, 128))
    return out.reshape(np_, ch)


def _pool_body(h_ref, b_ref, wg1_ref, bg1_ref, wg2_ref, bg2_ref,
               wm1_ref, bm1_ref, wm2_ref, bm2_ref, wm3_ref, bm3_ref,
               wm4_ref, bm4_ref, wm5_ref, bm5_ref, o_ref):
    batch = b_ref[...]
    valid = batch < _G
    h = jnp.where(valid, h_ref[...], 0.0)
    gate = jnp.maximum(
        jnp.dot(h, wg1_ref[...], preferred_element_type=jnp.float32)
        + bg1_ref[...], 0.0)
    gate = (jnp.dot(gate, wg2_ref[...], preferred_element_type=jnp.float32)
            + bg2_ref[...])
    gate = jnp.where(valid, gate, -1e30)

    gids = jax.lax.broadcasted_iota(jnp.int32, (1, _G), 1)
    onehot = (batch == gids).astype(jnp.float32)
    masked = jnp.where(onehot > 0, gate, -1e30)
    gm = jnp.max(masked, axis=0, keepdims=True)
    gm_row = jax.lax.dot_general(
        onehot, gm, (((1,), (1,)), ((), ())),
        preferred_element_type=jnp.float32)
    ge = jnp.exp(gate - gm_row)
    gs = jax.lax.dot_general(
        onehot, ge, (((0,), (0,)), ((), ())),
        preferred_element_type=jnp.float32)
    gs_row = jnp.dot(onehot, gs, preferred_element_type=jnp.float32)
    attw = ge / (gs_row + 1e-16)
    gemb = jax.lax.dot_general(
        onehot, attw * h, (((0,), (0,)), ((), ())),
        preferred_element_type=jnp.float32)

    z = jnp.maximum(jnp.dot(gemb, wm1_ref[...],
                            preferred_element_type=jnp.float32)
                    + bm1_ref[...], 0.0)
    z = jnp.maximum(jnp.dot(z, wm2_ref[...],
                            preferred_element_type=jnp.float32)
                    + bm2_ref[...], 0.0)
    z = jnp.maximum(jnp.dot(z, wm3_ref[...],
                            preferred_element_type=jnp.float32)
                    + bm3_ref[...], 0.0)
    z = jnp.maximum(jnp.dot(z, wm4_ref[...],
                            preferred_element_type=jnp.float32)
                    + bm4_ref[...], 0.0)
    o_ref[...] = (jnp.dot(z, wm5_ref[...],
                          preferred_element_type=jnp.float32) + bm5_ref[...])


def kernel(x, edge_index, batch, W1l, W1r, a1, b1, W2l, W2r, a2, b2,
           W3l, W3r, a3, b3, Wg1, bg1, Wg2, bg2, Wm1, bm1, Wm2, bm2,
           Wm3, bm3, Wm4, bm4, Wm5, bm5):
    n = x.shape[0]
    np_ = ((n + 255) // 256) * 256
    loop = jnp.arange(n, dtype=edge_index.dtype)
    src = jnp.concatenate([edge_index[0], loop])
    dst = jnp.concatenate([edge_index[1], loop])
    order = jnp.argsort(dst)
    src_s = src[order]
    dst_s = dst[order]

    h = jnp.pad(x, ((0, np_ - n), (0, 0)))
    h = _gat_layer(h, src_s, dst_s, W1l, W1r, a1, b1)
    h = _gat_layer(h, src_s, dst_s, W2l, W2r, a2, b2)
    h = _gat_layer(h, src_s, dst_s, W3l, W3r, a3, b3)

    bp = jnp.concatenate(
        [batch, jnp.full((np_ - n,), _G, batch.dtype)]).reshape(np_, 1)
    return pl.pallas_call(
        _pool_body,
        out_shape=jax.ShapeDtypeStruct((_G, 1), jnp.float32),
    )(h, bp, Wg1, bg1.reshape(1, -1), Wg2, bg2.reshape(1, -1),
      Wm1, bm1.reshape(1, -1), Wm2, bm2.reshape(1, -1),
      Wm3, bm3.reshape(1, -1), Wm4, bm4.reshape(1, -1),
      Wm5, bm5.reshape(1, -1))


# speedup vs baseline: 2.8061x; 2.8061x over previous
import jax
import jax.numpy as jnp
from jax.experimental import pallas as pl
from jax.experimental.pallas import tpu as pltpu

_G = 64


def _mm_body(a_ref, w_ref, o_ref):
    o_ref[...] = jnp.dot(a_ref[...], w_ref[...],
                         preferred_element_type=jnp.float32)


def _matmul(a, w, tm=256, tn=512):
    m, k = a.shape
    _, n = w.shape
    return pl.pallas_call(
        _mm_body,
        grid=(m // tm, n // tn),
        in_specs=[
            pl.BlockSpec((tm, k), lambda i, j: (i, 0)),
            pl.BlockSpec((k, tn), lambda i, j: (0, j)),
        ],
        out_specs=pl.BlockSpec((tm, tn), lambda i, j: (i, j)),
        out_shape=jax.ShapeDtypeStruct((m, n), jnp.float32),
    )(a, w)


_K = 8
_TILE = 64


def _edge_body(src_ref, dst_ref, nval_ref, xl_refs, xr_ref, att_ref, b_ref,
               o_ref, acc_ref, m_ref, s_ref):
    c = pl.program_id(0)
    ne = pl.num_programs(0) * _K
    e0 = c * _K
    heads = att_ref.shape[0]
    att = att_ref[...]
    nv = nval_ref[c]

    m_c = m_ref[...]
    s_c = s_ref[...]
    acc_c = acc_ref[...]
    neg = jnp.full((heads, 1, 128), -1e30, jnp.float32)

    for i in range(_K):
        ei = e0 + i
        d = dst_ref[ei]
        if i == 0:
            d_prev = dst_ref[jnp.maximum(ei - 1, 0)]
            first = jnp.logical_or(ei == 0, d != d_prev)
        else:
            first = d != dst_ref[ei - 1]
        last = jnp.logical_or(ei == ne - 1, d != dst_ref[jnp.minimum(ei + 1, ne - 1)])
        dloc = d - (dst_ref[e0] // _TILE) * _TILE

        xj = xl_refs[i][0]
        xi = xr_ref[pl.ds(dloc, 1)][0]
        z = xj + xi
        za = jnp.where(z >= 0, z, 0.2 * z)
        alpha = jnp.sum(za * att, axis=(1, 2), keepdims=True)
        alpha = jnp.where(i < nv, alpha, -1e30)
        alpha = jnp.broadcast_to(alpha, (heads, 1, 128))

        m_old = jnp.where(first, neg, m_c)
        s_old = jnp.where(first, jnp.zeros_like(neg), s_c)
        acc_old = jnp.where(first, jnp.zeros_like(xj), acc_c)

        m_c = jnp.maximum(m_old, alpha)
        corr = jnp.exp(m_old - m_c)
        w_e = jnp.exp(alpha - m_c)
        s_c = s_old * corr + w_e
        acc_c = acc_old * corr + xj * w_e

        out = jnp.mean(acc_c / s_c, axis=0, keepdims=True)
        out = jnp.maximum(out + b_ref[...], 0.0)

        @pl.when(last)
        def _(out=out, dloc=dloc):
            o_ref[pl.ds(dloc, 1)] = out

    m_ref[...] = m_c
    s_ref[...] = s_c
    acc_ref[...] = acc_c


def _gat_layer(h, src_p, dst_p, nval, wl, wr, att, bias):
    np_, _ = h.shape
    heads, ch = att.shape
    hh = ch // 128
    lr = _matmul(h, jnp.concatenate([wl, wr], axis=1))
    xl = lr[:, :heads * ch].reshape(np_, heads, hh, 128)
    xr = lr[:, heads * ch:].reshape(np_, heads, hh, 128)
    nchunks = nval.shape[0]

    def _body(src_ref, dst_ref, nval_ref, *refs):
        _edge_body(src_ref, dst_ref, nval_ref, refs[:_K], refs[_K],
                   refs[_K + 1], refs[_K + 2], refs[_K + 3], refs[_K + 4],
                   refs[_K + 5], refs[_K + 6])

    def _xl_map(i):
        return lambda c, s, d, nv: (s[c * _K + i], 0, 0, 0)

    out = pl.pallas_call(
        _body,
        grid_spec=pltpu.PrefetchScalarGridSpec(
            num_scalar_prefetch=3,
            grid=(nchunks,),
            in_specs=(
                [pl.BlockSpec((1, heads, hh, 128), _xl_map(i))
                 for i in range(_K)]
                + [pl.BlockSpec((_TILE, heads, hh, 128),
                                lambda c, s, d, nv: (d[c * _K] // _TILE,
                                                     0, 0, 0)),
                   pl.BlockSpec((heads, hh, 128),
                                lambda c, s, d, nv: (0, 0, 0)),
                   pl.BlockSpec((1, hh, 128),
                                lambda c, s, d, nv: (0, 0, 0))]
            ),
            out_specs=pl.BlockSpec((_TILE, hh, 128),
                                   lambda c, s, d, nv: (d[c * _K] // _TILE,
                                                        0, 0)),
            scratch_shapes=[
                pltpu.VMEM((heads, hh, 128), jnp.float32),
                pltpu.VMEM((heads, 1, 128), jnp.float32),
                pltpu.VMEM((heads, 1, 128), jnp.float32),
            ],
        ),
        out_shape=jax.ShapeDtypeStruct((np_, hh, 128), jnp.float32),
    )(src_p, dst_p, nval, *([xl] * _K), xr,
      att.reshape(heads, hh, 128), bias.reshape(1, hh, 128))
    return out.reshape(np_, ch)


def _pool_body(h_ref, b_ref, wg1_ref, bg1_ref, wg2_ref, bg2_ref,
               wm1_ref, bm1_ref, wm2_ref, bm2_ref, wm3_ref, bm3_ref,
               wm4_ref, bm4_ref, wm5_ref, bm5_ref, o_ref):
    batch = b_ref[...]
    valid = batch < _G
    h = jnp.where(valid, h_ref[...], 0.0)
    gate = jnp.maximum(
        jnp.dot(h, wg1_ref[...], preferred_element_type=jnp.float32)
        + bg1_ref[...], 0.0)
    gate = (jnp.dot(gate, wg2_ref[...], preferred_element_type=jnp.float32)
            + bg2_ref[...])
    gate = jnp.where(valid, gate, -1e30)

    gids = jax.lax.broadcasted_iota(jnp.int32, (1, _G), 1)
    onehot = (batch == gids).astype(jnp.float32)
    masked = jnp.where(onehot > 0, gate, -1e30)
    gm = jnp.max(masked, axis=0, keepdims=True)
    gm_row = jax.lax.dot_general(
        onehot, gm, (((1,), (1,)), ((), ())),
        preferred_element_type=jnp.float32)
    ge = jnp.exp(gate - gm_row)
    gs = jax.lax.dot_general(
        onehot, ge, (((0,), (0,)), ((), ())),
        preferred_element_type=jnp.float32)
    gs_row = jnp.dot(onehot, gs, preferred_element_type=jnp.float32)
    attw = ge / (gs_row + 1e-16)
    gemb = jax.lax.dot_general(
        onehot, attw * h, (((0,), (0,)), ((), ())),
        preferred_element_type=jnp.float32)

    z = jnp.maximum(jnp.dot(gemb, wm1_ref[...],
                            preferred_element_type=jnp.float32)
                    + bm1_ref[...], 0.0)
    z = jnp.maximum(jnp.dot(z, wm2_ref[...],
                            preferred_element_type=jnp.float32)
                    + bm2_ref[...], 0.0)
    z = jnp.maximum(jnp.dot(z, wm3_ref[...],
                            preferred_element_type=jnp.float32)
                    + bm3_ref[...], 0.0)
    z = jnp.maximum(jnp.dot(z, wm4_ref[...],
                            preferred_element_type=jnp.float32)
                    + bm4_ref[...], 0.0)
    o_ref[...] = (jnp.dot(z, wm5_ref[...],
                          preferred_element_type=jnp.float32) + bm5_ref[...])


def kernel(x, edge_index, batch, W1l, W1r, a1, b1, W2l, W2r, a2, b2,
           W3l, W3r, a3, b3, Wg1, bg1, Wg2, bg2, Wm1, bm1, Wm2, bm2,
           Wm3, bm3, Wm4, bm4, Wm5, bm5):
    n = x.shape[0]
    np_ = ((n + 255) // 256) * 256
    loop = jnp.arange(n, dtype=edge_index.dtype)
    src = jnp.concatenate([edge_index[0], loop])
    dst = jnp.concatenate([edge_index[1], loop])
    order = jnp.argsort(dst)
    src_s = src[order]
    dst_s = dst[order]

    e_tot = src_s.shape[0]
    t = (n + _TILE - 1) // _TILE
    tile_id = dst_s // _TILE
    counts = jnp.bincount(tile_id, length=t)
    tile_start = jnp.cumsum(counts) - counts
    nc = (counts + _K - 1) // _K
    nc_cum = jnp.cumsum(nc) - nc
    nchunks = e_tot // _K + t
    cid = jnp.arange(nchunks)
    t_of_c = jnp.searchsorted(nc_cum, cid, side='right') - 1
    base = tile_start[t_of_c] + (cid - nc_cum[t_of_c]) * _K
    tile_end = tile_start[t_of_c] + counts[t_of_c]
    nval = jnp.clip(tile_end - base, 0, _K).astype(jnp.int32)
    j = base[:, None] + jnp.arange(_K)[None, :]
    j = jnp.minimum(j, (tile_end - 1)[:, None])
    src_p = src_s[j].reshape(-1)
    dst_p = dst_s[j].reshape(-1)

    h = jnp.pad(x, ((0, np_ - n), (0, 0)))
    h = _gat_layer(h, src_p, dst_p, nval, W1l, W1r, a1, b1)
    h = _gat_layer(h, src_p, dst_p, nval, W2l, W2r, a2, b2)
    h = _gat_layer(h, src_p, dst_p, nval, W3l, W3r, a3, b3)

    bp = jnp.concatenate(
        [batch, jnp.full((np_ - n,), _G, batch.dtype)]).reshape(np_, 1)
    return pl.pallas_call(
        _pool_body,
        out_shape=jax.ShapeDtypeStruct((_G, 1), jnp.float32),
    )(h, bp, Wg1, bg1.reshape(1, -1), Wg2, bg2.reshape(1, -1),
      Wm1, bm1.reshape(1, -1), Wm2, bm2.reshape(1, -1),
      Wm3, bm3.reshape(1, -1), Wm4, bm4.reshape(1, -1),
      Wm5, bm5.reshape(1, -1))


# speedup vs baseline: 2.9486x; 1.0508x over previous
import jax
import jax.numpy as jnp
from jax.experimental import pallas as pl
from jax.experimental.pallas import tpu as pltpu

_G = 64


def _mm_body(a_ref, w_ref, o_ref):
    o_ref[...] = jnp.dot(a_ref[...], w_ref[...],
                         preferred_element_type=jnp.float32)


def _matmul(a, w, tm=256, tn=512):
    m, k = a.shape
    _, n = w.shape
    return pl.pallas_call(
        _mm_body,
        grid=(m // tm, n // tn),
        in_specs=[
            pl.BlockSpec((tm, k), lambda i, j: (i, 0)),
            pl.BlockSpec((k, tn), lambda i, j: (0, j)),
        ],
        out_specs=pl.BlockSpec((tm, tn), lambda i, j: (i, j)),
        out_shape=jax.ShapeDtypeStruct((m, n), jnp.float32),
    )(a, w)


_K = 8
_TILE = 64


def _edge_body(src_ref, dst_ref, nval_ref, xl_refs, xr_ref, att_ref, b_ref,
               o_ref, acc_ref, m_ref, s_ref):
    c = pl.program_id(0)
    ne = pl.num_programs(0) * _K
    e0 = c * _K
    heads = att_ref.shape[0]
    att = att_ref[...]
    nv = nval_ref[c]

    m_c = m_ref[...]
    s_c = s_ref[...]
    acc_c = acc_ref[...]
    neg = jnp.full((heads, 1, 128), -1e30, jnp.float32)

    for i in range(_K):
        ei = e0 + i
        d = dst_ref[ei]
        if i == 0:
            d_prev = dst_ref[jnp.maximum(ei - 1, 0)]
            first = jnp.logical_or(ei == 0, d != d_prev)
        else:
            first = d != dst_ref[ei - 1]
        last = jnp.logical_or(ei == ne - 1, d != dst_ref[jnp.minimum(ei + 1, ne - 1)])
        dloc = d - (dst_ref[e0] // _TILE) * _TILE

        xj = xl_refs[i][0]
        xi = xr_ref[pl.ds(dloc, 1)][0]
        z = xj + xi
        za = jnp.where(z >= 0, z, 0.2 * z)
        alpha = jnp.sum(za * att, axis=(1, 2), keepdims=True)
        alpha = jnp.where(i < nv, alpha, -1e30)
        alpha = jnp.broadcast_to(alpha, (heads, 1, 128))

        m_old = jnp.where(first, neg, m_c)
        s_old = jnp.where(first, jnp.zeros_like(neg), s_c)
        acc_old = jnp.where(first, jnp.zeros_like(xj), acc_c)

        m_c = jnp.maximum(m_old, alpha)
        corr = jnp.exp(m_old - m_c)
        w_e = jnp.exp(alpha - m_c)
        s_c = s_old * corr + w_e
        acc_c = acc_old * corr + xj * w_e

        @pl.when(last)
        def _(acc_c=acc_c, s_c=s_c, dloc=dloc):
            out = jnp.mean(acc_c * (1.0 / s_c), axis=0, keepdims=True)
            o_ref[pl.ds(dloc, 1)] = jnp.maximum(out + b_ref[...], 0.0)

    m_ref[...] = m_c
    s_ref[...] = s_c
    acc_ref[...] = acc_c


def _gat_layer(h, src_p, dst_p, nval, wl, wr, att, bias):
    np_, _ = h.shape
    heads, ch = att.shape
    hh = ch // 128
    lr = _matmul(h, jnp.concatenate([wl, wr], axis=1))
    xl = lr[:, :heads * ch].reshape(np_, heads, hh, 128)
    xr = lr[:, heads * ch:].reshape(np_, heads, hh, 128)
    nchunks = nval.shape[0]

    def _body(src_ref, dst_ref, nval_ref, *refs):
        _edge_body(src_ref, dst_ref, nval_ref, refs[:_K], refs[_K],
                   refs[_K + 1], refs[_K + 2], refs[_K + 3], refs[_K + 4],
                   refs[_K + 5], refs[_K + 6])

    def _xl_map(i):
        return lambda c, s, d, nv: (s[c * _K + i], 0, 0, 0)

    out = pl.pallas_call(
        _body,
        grid_spec=pltpu.PrefetchScalarGridSpec(
            num_scalar_prefetch=3,
            grid=(nchunks,),
            in_specs=(
                [pl.BlockSpec((1, heads, hh, 128), _xl_map(i))
                 for i in range(_K)]
                + [pl.BlockSpec((_TILE, heads, hh, 128),
                                lambda c, s, d, nv: (d[c * _K] // _TILE,
                                                     0, 0, 0)),
                   pl.BlockSpec((heads, hh, 128),
                                lambda c, s, d, nv: (0, 0, 0)),
                   pl.BlockSpec((1, hh, 128),
                                lambda c, s, d, nv: (0, 0, 0))]
            ),
            out_specs=pl.BlockSpec((_TILE, hh, 128),
                                   lambda c, s, d, nv: (d[c * _K] // _TILE,
                                                        0, 0)),
            scratch_shapes=[
                pltpu.VMEM((heads, hh, 128), jnp.float32),
                pltpu.VMEM((heads, 1, 128), jnp.float32),
                pltpu.VMEM((heads, 1, 128), jnp.float32),
            ],
        ),
        out_shape=jax.ShapeDtypeStruct((np_, hh, 128), jnp.float32),
    )(src_p, dst_p, nval, *([xl] * _K), xr,
      att.reshape(heads, hh, 128), bias.reshape(1, hh, 128))
    return out.reshape(np_, ch)


def _pool_body(h_ref, b_ref, wg1_ref, bg1_ref, wg2_ref, bg2_ref,
               wm1_ref, bm1_ref, wm2_ref, bm2_ref, wm3_ref, bm3_ref,
               wm4_ref, bm4_ref, wm5_ref, bm5_ref, o_ref):
    batch = b_ref[...]
    valid = batch < _G
    h = jnp.where(valid, h_ref[...], 0.0)
    gate = jnp.maximum(
        jnp.dot(h, wg1_ref[...], preferred_element_type=jnp.float32)
        + bg1_ref[...], 0.0)
    gate = (jnp.dot(gate, wg2_ref[...], preferred_element_type=jnp.float32)
            + bg2_ref[...])
    gate = jnp.where(valid, gate, -1e30)

    gids = jax.lax.broadcasted_iota(jnp.int32, (1, _G), 1)
    onehot = (batch == gids).astype(jnp.float32)
    masked = jnp.where(onehot > 0, gate, -1e30)
    gm = jnp.max(masked, axis=0, keepdims=True)
    gm_row = jax.lax.dot_general(
        onehot, gm, (((1,), (1,)), ((), ())),
        preferred_element_type=jnp.float32)
    ge = jnp.exp(gate - gm_row)
    gs = jax.lax.dot_general(
        onehot, ge, (((0,), (0,)), ((), ())),
        preferred_element_type=jnp.float32)
    gs_row = jnp.dot(onehot, gs, preferred_element_type=jnp.float32)
    attw = ge / (gs_row + 1e-16)
    gemb = jax.lax.dot_general(
        onehot, attw * h, (((0,), (0,)), ((), ())),
        preferred_element_type=jnp.float32)

    z = jnp.maximum(jnp.dot(gemb, wm1_ref[...],
                            preferred_element_type=jnp.float32)
                    + bm1_ref[...], 0.0)
    z = jnp.maximum(jnp.dot(z, wm2_ref[...],
                            preferred_element_type=jnp.float32)
                    + bm2_ref[...], 0.0)
    z = jnp.maximum(jnp.dot(z, wm3_ref[...],
                            preferred_element_type=jnp.float32)
                    + bm3_ref[...], 0.0)
    z = jnp.maximum(jnp.dot(z, wm4_ref[...],
                            preferred_element_type=jnp.float32)
                    + bm4_ref[...], 0.0)
    o_ref[...] = (jnp.dot(z, wm5_ref[...],
                          preferred_element_type=jnp.float32) + bm5_ref[...])


def kernel(x, edge_index, batch, W1l, W1r, a1, b1, W2l, W2r, a2, b2,
           W3l, W3r, a3, b3, Wg1, bg1, Wg2, bg2, Wm1, bm1, Wm2, bm2,
           Wm3, bm3, Wm4, bm4, Wm5, bm5):
    n = x.shape[0]
    np_ = ((n + 255) // 256) * 256
    loop = jnp.arange(n, dtype=edge_index.dtype)
    src = jnp.concatenate([edge_index[0], loop])
    dst = jnp.concatenate([edge_index[1], loop])
    order = jnp.argsort(dst)
    src_s = src[order]
    dst_s = dst[order]

    e_tot = src_s.shape[0]
    t = (n + _TILE - 1) // _TILE
    tile_id = dst_s // _TILE
    counts = jnp.bincount(tile_id, length=t)
    tile_start = jnp.cumsum(counts) - counts
    nc = (counts + _K - 1) // _K
    nc_cum = jnp.cumsum(nc) - nc
    nchunks = e_tot // _K + t
    cid = jnp.arange(nchunks)
    t_of_c = jnp.searchsorted(nc_cum, cid, side='right') - 1
    base = tile_start[t_of_c] + (cid - nc_cum[t_of_c]) * _K
    tile_end = tile_start[t_of_c] + counts[t_of_c]
    nval = jnp.clip(tile_end - base, 0, _K).astype(jnp.int32)
    j = base[:, None] + jnp.arange(_K)[None, :]
    j = jnp.minimum(j, (tile_end - 1)[:, None])
    src_p = src_s[j].reshape(-1)
    dst_p = dst_s[j].reshape(-1)

    h = jnp.pad(x, ((0, np_ - n), (0, 0)))
    h = _gat_layer(h, src_p, dst_p, nval, W1l, W1r, a1, b1)
    h = _gat_layer(h, src_p, dst_p, nval, W2l, W2r, a2, b2)
    h = _gat_layer(h, src_p, dst_p, nval, W3l, W3r, a3, b3)

    bp = jnp.concatenate(
        [batch, jnp.full((np_ - n,), _G, batch.dtype)]).reshape(np_, 1)
    return pl.pallas_call(
        _pool_body,
        out_shape=jax.ShapeDtypeStruct((_G, 1), jnp.float32),
    )(h, bp, Wg1, bg1.reshape(1, -1), Wg2, bg2.reshape(1, -1),
      Wm1, bm1.reshape(1, -1), Wm2, bm2.reshape(1, -1),
      Wm3, bm3.reshape(1, -1), Wm4, bm4.reshape(1, -1),
      Wm5, bm5.reshape(1, -1))
